# baseline (speedup 1.0000x reference)
import functools

import jax
import jax.numpy as jnp
from jax import lax
from jax.experimental import pallas as pl
from jax.experimental.pallas import tpu as pltpu
from jax.experimental.pallas import tpu_sc as plsc

N = 10000
K = 32
D = 128
O = 128
C = 128

NW = 32
NPW = 320
NPAD = NW * NPW
CN = 2
CHUNK = CN * K
NCH = NPW // CN
NV = O // 16
EROWS = N * K // 128
EPAD = NPAD * K // 128

BN = 400


def _film_body(cond_ref, v_ref, g_ref, b2_ref, x_ref, w_ref, bf_ref,
               wv_ref, wp_ref, o_ref, ew_ref, gb_ref):
  dn = (((1,), (1,)), ((), ()))

  @pl.when(pl.program_id(0) == 0)
  def _():
    v = v_ref[...]
    ssq = jnp.sum(v * v, axis=1, keepdims=True)
    vs = v * (g_ref[...] * lax.rsqrt(ssq))
    gb_ref[...] = lax.dot_general(
        cond_ref[...], vs, dn, preferred_element_type=jnp.float32,
        precision=lax.Precision.HIGHEST) + b2_ref[...]
    ew_ref[pl.ds(0, EROWS)] = wv_ref[...] * wp_ref[...]

  h = lax.dot_general(x_ref[...], w_ref[...], dn,
                      preferred_element_type=jnp.float32,
                      precision=lax.Precision.HIGHEST) + bf_ref[...]
  mu = jnp.mean(h, axis=1, keepdims=True)
  hc = h - mu
  var = jnp.mean(hc * hc, axis=1, keepdims=True)
  hn = hc * lax.rsqrt(var + 1e-5)
  gamma = gb_ref[:, :O] + 1.0
  beta = gb_ref[:, O:]
  o_ref[...] = jnp.maximum(hn * gamma + beta, 0.0)


def _film_tc(x2, cond2, v, g2, b2, w, bf2, wv2, wp2):
  grid = N // BN
  return pl.pallas_call(
      _film_body,
      grid=(grid,),
      in_specs=[
          pl.BlockSpec((1, C), lambda i: (0, 0)),
          pl.BlockSpec((2 * O, C), lambda i: (0, 0)),
          pl.BlockSpec((2 * O, 1), lambda i: (0, 0)),
          pl.BlockSpec((1, 2 * O), lambda i: (0, 0)),
          pl.BlockSpec((BN, D), lambda i: (i, 0)),
          pl.BlockSpec((O, D), lambda i: (0, 0)),
          pl.BlockSpec((1, O), lambda i: (0, 0)),
          pl.BlockSpec((EROWS, 128), lambda i: (0, 0)),
          pl.BlockSpec((EROWS, 128), lambda i: (0, 0)),
      ],
      out_specs=[
          pl.BlockSpec((BN, O), lambda i: (i, 0)),
          pl.BlockSpec((EPAD, 128), lambda i: (0, 0)),
      ],
      out_shape=[
          jax.ShapeDtypeStruct((NPAD, O), jnp.float32),
          jax.ShapeDtypeStruct((EPAD, 128), jnp.float32),
      ],
      scratch_shapes=[pltpu.VMEM((1, 2 * O), jnp.float32)],
  )(cond2, v, g2, b2, x2, w, bf2, wv2, wp2)


_SC_MESH = plsc.VectorSubcoreMesh(core_axis_name="c", subcore_axis_name="s")


@functools.partial(
    pl.kernel,
    out_type=jax.ShapeDtypeStruct((NPAD, O), jnp.float32),
    mesh=_SC_MESH,
    scratch_types=[
        pltpu.VMEM((NCH // 2, 2 * CHUNK), jnp.int32),
        pltpu.VMEM((NCH // 2, 2 * CHUNK), jnp.float32),
        pltpu.VMEM((CHUNK, O), jnp.float32),
        pltpu.VMEM((CHUNK, O), jnp.float32),
        pltpu.VMEM((2 * CN, O), jnp.float32),
        pltpu.VMEM((2 * CN, O), jnp.float32),
        pltpu.VMEM((CN, O), jnp.float32),
        pltpu.VMEM((CN, O), jnp.float32),
        pltpu.VMEM_SHARED((NPAD, O), jnp.float32),
        pltpu.SemaphoreType.DMA,
        pltpu.SemaphoreType.DMA,
        pltpu.SemaphoreType.DMA,
        pltpu.SemaphoreType.DMA,
        pltpu.SemaphoreType.DMA,
        pltpu.SemaphoreType.DMA,
    ],
)
def _agg_sc(feats_hbm, idx_hbm, ew_hbm, out_hbm,
            idx_v, ew_v, rows0, rows1, s0, s1, nb0, nb1, shared,
            sem0, sem1, ssem0, ssem1, wsem0, wsem1):
  nc = _SC_MESH.num_cores
  ns = _SC_MESH.num_subcores
  sid = lax.axis_index("s")
  wid = sid * nc + lax.axis_index("c")
  base = wid * NPW

  seg = NPAD // ns
  pltpu.sync_copy(feats_hbm.at[pl.ds(sid * seg, seg)],
                  shared.at[pl.ds(sid * seg, seg)])

  nr = NCH // 2
  pltpu.sync_copy(idx_hbm.at[pl.ds(wid * nr, nr)], idx_v)
  pltpu.sync_copy(ew_hbm.at[pl.ds(wid * nr, nr)], ew_v)
  plsc.subcore_barrier()

  def _start(c, buf, sem):
    idx = idx_v.at[c // 2, pl.ds((c % 2) * CHUNK, CHUNK)]
    pltpu.async_copy(shared.at[idx], buf, sem)

  def _wait(buf, sem):
    pltpu.make_async_copy(shared.at[idx_v.at[0, pl.ds(0, CHUNK)]], buf,
                          sem).wait()

  def _sstart(c, sbuf, ssem):
    pltpu.async_copy(shared.at[pl.ds(base + c * CN, 2 * CN)], sbuf, ssem)

  def _swait(sbuf, ssem):
    pltpu.make_async_copy(shared.at[pl.ds(0, 2 * CN)], sbuf, ssem).wait()

  def _wwait(nbuf, wsem):
    pltpu.make_async_copy(out_hbm.at[pl.ds(0, CN)], nbuf, wsem).wait()

  def _compute(c, buf, sbuf, srow, nbuf, wsem):
    for n in range(CN):
      acc = [sbuf[srow + n, pl.ds(d * 16, 16)] for d in range(NV)]
      for half in range(K // 16):
        woff = (c % 2) * CHUNK + n * K + half * 16
        wvec = ew_v[c // 2, pl.ds(woff, 16)]
        for kk in range(16):
          e = n * K + half * 16 + kk
          w = lax.broadcast(wvec[kk], (16,))
          for d in range(NV):
            acc[d] = acc[d] + w * buf[e, pl.ds(d * 16, 16)]
      for d in range(NV):
        nbuf[n, pl.ds(d * 16, 16)] = jnp.maximum(acc[d], 0.0)
    pltpu.async_copy(nbuf, out_hbm.at[pl.ds(base + c * CN, CN)], wsem)

  _start(0, rows0, sem0)
  _start(1, rows1, sem1)
  _sstart(0, s0, ssem0)
  _sstart(2, s1, ssem1)

  def _body(j, carry):
    c0 = 4 * j

    _wait(rows0, sem0)
    _swait(s0, ssem0)

    @pl.when(c0 >= 2)
    def _():
      _wwait(nb0, wsem0)

    _compute(c0, rows0, s0, 0, nb0, wsem0)
    _start(c0 + 2, rows0, sem0)

    _wait(rows1, sem1)

    @pl.when(c0 >= 2)
    def _():
      _wwait(nb1, wsem1)

    _compute(c0 + 1, rows1, s0, CN, nb1, wsem1)
    _start(c0 + 3, rows1, sem1)

    @pl.when(c0 + 4 < NCH)
    def _():
      _sstart(c0 + 4, s0, ssem0)

    _wait(rows0, sem0)
    _swait(s1, ssem1)
    _wwait(nb0, wsem0)
    _compute(c0 + 2, rows0, s1, 0, nb0, wsem0)

    @pl.when(c0 + 4 < NCH)
    def _():
      _start(c0 + 4, rows0, sem0)

    _wait(rows1, sem1)
    _wwait(nb1, wsem1)
    _compute(c0 + 3, rows1, s1, CN, nb1, wsem1)

    @pl.when(c0 + 4 < NCH)
    def _():
      _start(c0 + 5, rows1, sem1)

      @pl.when(c0 + 6 < NCH)
      def _():
        _sstart(c0 + 6, s1, ssem1)

    return carry

  lax.fori_loop(0, NCH // 4, _body, 0)

  _wwait(nb0, wsem0)
  _wwait(nb1, wsem1)


def kernel(x, cond, idx_j, w_val, w_param, v_cond, g_cond, b_cond,
           W_film, b_film):
  feats, ew = _film_tc(
      x.reshape(N, D),
      cond.reshape(1, C),
      v_cond,
      g_cond.reshape(2 * O, 1),
      b_cond.reshape(1, 2 * O),
      W_film,
      b_film.reshape(1, O),
      w_val.reshape(EROWS, 128),
      w_param.reshape(EROWS, 128),
  )

  pad_e = (NPAD - N) * K
  idxp = jnp.pad(idx_j.astype(jnp.int32), (0, pad_e)).reshape(EPAD, 128)

  out = _agg_sc(feats, idxp, ew)
  return out[:N].reshape(1, N, O)


# speedup vs baseline: 1.4430x; 1.4430x over previous
import functools

import jax
import jax.numpy as jnp
from jax import lax
from jax.experimental import pallas as pl
from jax.experimental.pallas import tpu as pltpu
from jax.experimental.pallas import tpu_sc as plsc

N = 10000
K = 32
D = 128
O = 128
C = 128

NW = 32
NPW = 320
NPAD = NW * NPW
CN = 2
CHUNK = CN * K
NCH = NPW // CN
NV = O // 16
EROWS = N * K // 128
EPAD = NPAD * K // 128

BN = 400


def _film_body(cond_ref, v_ref, g_ref, b2_ref, x_ref, w_ref, bf_ref,
               wv_ref, wp_ref, o_ref, ew_ref, gb_ref):
  dn = (((1,), (1,)), ((), ()))

  @pl.when(pl.program_id(0) == 0)
  def _():
    v = v_ref[...]
    ssq = jnp.sum(v * v, axis=1, keepdims=True)
    vs = v * (g_ref[...] * lax.rsqrt(ssq))
    gb_ref[...] = lax.dot_general(
        cond_ref[...], vs, dn, preferred_element_type=jnp.float32,
        precision=lax.Precision.HIGHEST) + b2_ref[...]
    ew_ref[pl.ds(0, EROWS)] = wv_ref[...] * wp_ref[...]

  h = lax.dot_general(x_ref[...], w_ref[...], dn,
                      preferred_element_type=jnp.float32,
                      precision=lax.Precision.HIGHEST) + bf_ref[...]
  mu = jnp.mean(h, axis=1, keepdims=True)
  hc = h - mu
  var = jnp.mean(hc * hc, axis=1, keepdims=True)
  hn = hc * lax.rsqrt(var + 1e-5)
  gamma = gb_ref[:, :O] + 1.0
  beta = gb_ref[:, O:]
  o_ref[...] = jnp.maximum(hn * gamma + beta, 0.0)


def _film_tc(x2, cond2, v, g2, b2, w, bf2, wv2, wp2):
  grid = N // BN
  return pl.pallas_call(
      _film_body,
      grid=(grid,),
      in_specs=[
          pl.BlockSpec((1, C), lambda i: (0, 0)),
          pl.BlockSpec((2 * O, C), lambda i: (0, 0)),
          pl.BlockSpec((2 * O, 1), lambda i: (0, 0)),
          pl.BlockSpec((1, 2 * O), lambda i: (0, 0)),
          pl.BlockSpec((BN, D), lambda i: (i, 0)),
          pl.BlockSpec((O, D), lambda i: (0, 0)),
          pl.BlockSpec((1, O), lambda i: (0, 0)),
          pl.BlockSpec((EROWS, 128), lambda i: (0, 0)),
          pl.BlockSpec((EROWS, 128), lambda i: (0, 0)),
      ],
      out_specs=[
          pl.BlockSpec((BN, O), lambda i: (i, 0)),
          pl.BlockSpec((EPAD, 128), lambda i: (0, 0)),
      ],
      out_shape=[
          jax.ShapeDtypeStruct((NPAD, O), jnp.float32),
          jax.ShapeDtypeStruct((EPAD, 128), jnp.float32),
      ],
      scratch_shapes=[pltpu.VMEM((1, 2 * O), jnp.float32)],
  )(cond2, v, g2, b2, x2, w, bf2, wv2, wp2)


_SC_MESH = plsc.VectorSubcoreMesh(core_axis_name="c", subcore_axis_name="s")


@functools.partial(
    pl.kernel,
    out_type=jax.ShapeDtypeStruct((NPAD, O), jnp.float32),
    mesh=_SC_MESH,
    scratch_types=[
        pltpu.VMEM((NCH // 2, 2 * CHUNK), jnp.int32),
        pltpu.VMEM((NCH // 2, 2 * CHUNK), jnp.float32),
        pltpu.VMEM((CHUNK, O), jnp.float32),
        pltpu.VMEM((CHUNK, O), jnp.float32),
        pltpu.VMEM((CN, O), jnp.float32),
        pltpu.VMEM((CN, O), jnp.float32),
        pltpu.VMEM((CN, O), jnp.float32),
        pltpu.VMEM((CN, O), jnp.float32),
        pltpu.VMEM_SHARED((NPAD, O), jnp.float32),
        pltpu.SemaphoreType.DMA,
        pltpu.SemaphoreType.DMA,
        pltpu.SemaphoreType.DMA,
        pltpu.SemaphoreType.DMA,
        pltpu.SemaphoreType.DMA,
        pltpu.SemaphoreType.DMA,
    ],
)
def _agg_sc(feats_hbm, idx_hbm, ew_hbm, out_hbm,
            idx_v, ew_v, rows0, rows1, s0, s1, nb0, nb1, shared,
            sem0, sem1, ssem0, ssem1, wsem0, wsem1):
  nc = _SC_MESH.num_cores
  ns = _SC_MESH.num_subcores
  sid = lax.axis_index("s")
  wid = sid * nc + lax.axis_index("c")
  base = wid * NPW

  seg = NPAD // ns
  pltpu.sync_copy(feats_hbm.at[pl.ds(sid * seg, seg)],
                  shared.at[pl.ds(sid * seg, seg)])

  nr = NCH // 2
  pltpu.sync_copy(idx_hbm.at[pl.ds(wid * nr, nr)], idx_v)
  pltpu.sync_copy(ew_hbm.at[pl.ds(wid * nr, nr)], ew_v)
  plsc.subcore_barrier()

  def _start(c, buf, sem):
    idx = idx_v.at[c // 2, pl.ds((c % 2) * CHUNK, CHUNK)]
    pltpu.async_copy(shared.at[idx], buf, sem)

  def _wait(buf, sem):
    pltpu.make_async_copy(shared.at[idx_v.at[0, pl.ds(0, CHUNK)]], buf,
                          sem).wait()

  def _sstart(c, sbuf, ssem):
    pltpu.async_copy(shared.at[pl.ds(base + c * CN, CN)], sbuf, ssem)

  def _swait(sbuf, ssem):
    pltpu.make_async_copy(shared.at[pl.ds(0, CN)], sbuf, ssem).wait()

  def _wwait(nbuf, wsem):
    pltpu.make_async_copy(out_hbm.at[pl.ds(0, CN)], nbuf, wsem).wait()

  def _compute(c, buf, sbuf, nbuf, wsem):
    for n in range(CN):
      acc = [sbuf[n, pl.ds(d * 16, 16)] for d in range(NV)]
      for half in range(K // 16):
        woff = (c % 2) * CHUNK + n * K + half * 16
        wvec = ew_v[c // 2, pl.ds(woff, 16)]
        for kk in range(16):
          e = n * K + half * 16 + kk
          w = lax.broadcast(wvec[kk], (16,))
          for d in range(NV):
            acc[d] = acc[d] + w * buf[e, pl.ds(d * 16, 16)]
      for d in range(NV):
        nbuf[n, pl.ds(d * 16, 16)] = jnp.maximum(acc[d], 0.0)
    pltpu.async_copy(nbuf, out_hbm.at[pl.ds(base + c * CN, CN)], wsem)

  _start(0, rows0, sem0)
  _sstart(0, s0, ssem0)

  def _body(i, carry):
    c0 = 2 * i
    c1 = c0 + 1
    _start(c1, rows1, sem1)
    _sstart(c1, s1, ssem1)
    _wait(rows0, sem0)
    _swait(s0, ssem0)

    @pl.when(c0 >= 2)
    def _():
      _wwait(nb0, wsem0)

    _compute(c0, rows0, s0, nb0, wsem0)

    @pl.when(c0 + 2 < NCH)
    def _():
      _start(c0 + 2, rows0, sem0)
      _sstart(c0 + 2, s0, ssem0)

    _wait(rows1, sem1)
    _swait(s1, ssem1)

    @pl.when(c0 >= 2)
    def _():
      _wwait(nb1, wsem1)

    _compute(c1, rows1, s1, nb1, wsem1)
    return carry

  lax.fori_loop(0, NCH // 2, _body, 0)

  _wwait(nb0, wsem0)
  _wwait(nb1, wsem1)


def kernel(x, cond, idx_j, w_val, w_param, v_cond, g_cond, b_cond,
           W_film, b_film):
  feats, ew = _film_tc(
      x.reshape(N, D),
      cond.reshape(1, C),
      v_cond,
      g_cond.reshape(2 * O, 1),
      b_cond.reshape(1, 2 * O),
      W_film,
      b_film.reshape(1, O),
      w_val.reshape(EROWS, 128),
      w_param.reshape(EROWS, 128),
  )

  pad_e = (NPAD - N) * K
  idxp = jnp.pad(idx_j.astype(jnp.int32), (0, pad_e)).reshape(EPAD, 128)

  out = _agg_sc(feats, idxp, ew)
  return out[:N].reshape(1, N, O)


# speedup vs baseline: 1.5391x; 1.0666x over previous
import functools

import jax
import jax.numpy as jnp
from jax import lax
from jax.experimental import pallas as pl
from jax.experimental.pallas import tpu as pltpu
from jax.experimental.pallas import tpu_sc as plsc

N = 10000
K = 32
D = 128
O = 128
C = 128

NW = 32
NPW = 320
NPAD = NW * NPW
CN = 2
CHUNK = CN * K
NCH = NPW // CN
NV = O // 16
EROWS = N * K // 128
EPAD = NPAD * K // 128

BN = 2000


def _film_body(cond_ref, v_ref, g_ref, b2_ref, x_ref, w_ref, bf_ref,
               wv_ref, wp_ref, o_ref, ew_ref, gb_ref):
  dn = (((1,), (1,)), ((), ()))

  @pl.when(pl.program_id(0) == 0)
  def _():
    v = v_ref[...]
    ssq = jnp.sum(v * v, axis=1, keepdims=True)
    vs = v * (g_ref[...] * lax.rsqrt(ssq))
    gb_ref[...] = lax.dot_general(
        cond_ref[...], vs, dn, preferred_element_type=jnp.float32,
        precision=lax.Precision.HIGHEST) + b2_ref[...]
    ew_ref[pl.ds(0, EROWS)] = wv_ref[...] * wp_ref[...]

  h = lax.dot_general(x_ref[...], w_ref[...], dn,
                      preferred_element_type=jnp.float32,
                      precision=lax.Precision.HIGHEST) + bf_ref[...]
  mu = jnp.mean(h, axis=1, keepdims=True)
  hc = h - mu
  var = jnp.mean(hc * hc, axis=1, keepdims=True)
  hn = hc * lax.rsqrt(var + 1e-5)
  gamma = gb_ref[:, :O] + 1.0
  beta = gb_ref[:, O:]
  o_ref[...] = jnp.maximum(hn * gamma + beta, 0.0)


def _film_tc(x2, cond2, v, g2, b2, w, bf2, wv2, wp2):
  grid = N // BN
  return pl.pallas_call(
      _film_body,
      grid=(grid,),
      in_specs=[
          pl.BlockSpec((1, C), lambda i: (0, 0)),
          pl.BlockSpec((2 * O, C), lambda i: (0, 0)),
          pl.BlockSpec((2 * O, 1), lambda i: (0, 0)),
          pl.BlockSpec((1, 2 * O), lambda i: (0, 0)),
          pl.BlockSpec((BN, D), lambda i: (i, 0)),
          pl.BlockSpec((O, D), lambda i: (0, 0)),
          pl.BlockSpec((1, O), lambda i: (0, 0)),
          pl.BlockSpec((EROWS, 128), lambda i: (0, 0)),
          pl.BlockSpec((EROWS, 128), lambda i: (0, 0)),
      ],
      out_specs=[
          pl.BlockSpec((BN, O), lambda i: (i, 0)),
          pl.BlockSpec((EPAD, 128), lambda i: (0, 0)),
      ],
      out_shape=[
          jax.ShapeDtypeStruct((NPAD, O), jnp.float32),
          jax.ShapeDtypeStruct((EPAD, 128), jnp.float32),
      ],
      scratch_shapes=[pltpu.VMEM((1, 2 * O), jnp.float32)],
  )(cond2, v, g2, b2, x2, w, bf2, wv2, wp2)


_SC_MESH = plsc.VectorSubcoreMesh(core_axis_name="c", subcore_axis_name="s")


@functools.partial(
    pl.kernel,
    out_type=jax.ShapeDtypeStruct((NPAD, O), jnp.float32),
    mesh=_SC_MESH,
    scratch_types=[
        pltpu.VMEM((NCH // 2, 2 * CHUNK), jnp.int32),
        pltpu.VMEM((NCH // 2, 2 * CHUNK), jnp.float32),
        pltpu.VMEM((CHUNK, O), jnp.float32),
        pltpu.VMEM((CHUNK, O), jnp.float32),
        pltpu.VMEM((CN, O), jnp.float32),
        pltpu.VMEM((CN, O), jnp.float32),
        pltpu.VMEM((CN, O), jnp.float32),
        pltpu.VMEM((CN, O), jnp.float32),
        pltpu.VMEM_SHARED((NPAD, O), jnp.float32),
        pltpu.SemaphoreType.DMA,
        pltpu.SemaphoreType.DMA,
        pltpu.SemaphoreType.DMA,
        pltpu.SemaphoreType.DMA,
        pltpu.SemaphoreType.DMA,
        pltpu.SemaphoreType.DMA,
    ],
)
def _agg_sc(feats_hbm, idx_hbm, ew_hbm, out_hbm,
            idx_v, ew_v, rows0, rows1, s0, s1, nb0, nb1, shared,
            sem0, sem1, ssem0, ssem1, wsem0, wsem1):
  nc = _SC_MESH.num_cores
  ns = _SC_MESH.num_subcores
  sid = lax.axis_index("s")
  wid = sid * nc + lax.axis_index("c")
  base = wid * NPW

  seg = NPAD // ns
  pltpu.sync_copy(feats_hbm.at[pl.ds(sid * seg, seg)],
                  shared.at[pl.ds(sid * seg, seg)])

  nr = NCH // 2
  pltpu.sync_copy(idx_hbm.at[pl.ds(wid * nr, nr)], idx_v)
  pltpu.sync_copy(ew_hbm.at[pl.ds(wid * nr, nr)], ew_v)
  plsc.subcore_barrier()

  def _start(c, buf, sem):
    idx = idx_v.at[c // 2, pl.ds((c % 2) * CHUNK, CHUNK)]
    pltpu.async_copy(shared.at[idx], buf, sem)

  def _wait(buf, sem):
    pltpu.make_async_copy(shared.at[idx_v.at[0, pl.ds(0, CHUNK)]], buf,
                          sem).wait()

  def _sstart(c, sbuf, ssem):
    pltpu.async_copy(shared.at[pl.ds(base + c * CN, CN)], sbuf, ssem)

  def _swait(sbuf, ssem):
    pltpu.make_async_copy(shared.at[pl.ds(0, CN)], sbuf, ssem).wait()

  def _wwait(nbuf, wsem):
    pltpu.make_async_copy(out_hbm.at[pl.ds(0, CN)], nbuf, wsem).wait()

  def _compute(c, buf, sbuf, nbuf, wsem):
    for n in range(CN):
      acc = [sbuf[n, pl.ds(d * 16, 16)] for d in range(NV)]
      for half in range(K // 16):
        woff = (c % 2) * CHUNK + n * K + half * 16
        wvec = ew_v[c // 2, pl.ds(woff, 16)]
        for kk in range(16):
          e = n * K + half * 16 + kk
          w = lax.broadcast(wvec[kk], (16,))
          for d in range(NV):
            acc[d] = acc[d] + w * buf[e, pl.ds(d * 16, 16)]
      for d in range(NV):
        nbuf[n, pl.ds(d * 16, 16)] = jnp.maximum(acc[d], 0.0)
    pltpu.async_copy(nbuf, out_hbm.at[pl.ds(base + c * CN, CN)], wsem)

  _start(0, rows0, sem0)
  _sstart(0, s0, ssem0)

  def _body(i, carry):
    c0 = 2 * i
    c1 = c0 + 1
    _start(c1, rows1, sem1)
    _sstart(c1, s1, ssem1)
    _wait(rows0, sem0)
    _swait(s0, ssem0)

    @pl.when(c0 >= 2)
    def _():
      _wwait(nb0, wsem0)

    _compute(c0, rows0, s0, nb0, wsem0)

    @pl.when(c0 + 2 < NCH)
    def _():
      _start(c0 + 2, rows0, sem0)
      _sstart(c0 + 2, s0, ssem0)

    _wait(rows1, sem1)
    _swait(s1, ssem1)

    @pl.when(c0 >= 2)
    def _():
      _wwait(nb1, wsem1)

    _compute(c1, rows1, s1, nb1, wsem1)
    return carry

  lax.fori_loop(0, NCH // 2, _body, 0)

  _wwait(nb0, wsem0)
  _wwait(nb1, wsem1)


def kernel(x, cond, idx_j, w_val, w_param, v_cond, g_cond, b_cond,
           W_film, b_film):
  feats, ew = _film_tc(
      x.reshape(N, D),
      cond.reshape(1, C),
      v_cond,
      g_cond.reshape(2 * O, 1),
      b_cond.reshape(1, 2 * O),
      W_film,
      b_film.reshape(1, O),
      w_val.reshape(EROWS, 128),
      w_param.reshape(EROWS, 128),
  )

  pad_e = (NPAD - N) * K
  idxp = jnp.pad(idx_j.astype(jnp.int32), (0, pad_e)).reshape(EPAD, 128)

  out = _agg_sc(feats, idxp, ew)
  return out[:N].reshape(1, N, O)
